# Initial kernel scaffold; baseline (speedup 1.0000x reference)
import jax
import jax.numpy as jnp
from jax.experimental import pallas as pl


def kernel(x, pos, edge_index_intra, edge_index_inter, batch, params):
    raise NotImplementedError("write your pallas kernel here")


# speedup vs baseline: 2.9329x; 2.9329x over previous
import functools

import jax
import jax.numpy as jnp
from jax import lax
from jax.experimental import pallas as pl
from jax.experimental.pallas import tpu as pltpu
from jax.experimental.pallas import tpu_sc as plsc

N, D, E, G = 10000, 128, 160000, 64

_NC, _NS = 2, 16
_CHUNK = 128
_NCHUNK = 80
_EPW = _CHUNK * _NCHUNK
_EPAD = _NS * _EPW
_NPAD = 10240
_NW = _NC * _NS
_NCHUNK_A = _EPAD // (_NW * _CHUNK)
_EPW_A = _NCHUNK_A * _CHUNK
_ROWS_PER_SUB = _NPAD // _NS
_ZROWS = 128


def _rbf(Dv, D_min=0.0, D_max=6.0, D_count=9):
    mu = jnp.linspace(D_min, D_max, D_count)
    sigma = (D_max - D_min) / D_count
    return jnp.exp(-(((Dv[..., None] - mu) / sigma) ** 2))


def _bn(h, g, b, eps=1e-5):
    m = h.mean(axis=0)
    v = h.var(axis=0)
    return g * (h - m) / jnp.sqrt(v + eps) + b


_DH = D // 2


def _sc_body(h0_hbm, h1_hbm, rad_hbm, row_hbm, col_hbm, out_hbm,
             row_v, col_v, hrows, rad, agg_sh, gsem, rsem, ssem):
    cid = lax.axis_index("c")
    sid = lax.axis_index("s")
    wid = sid * _NC + cid

    zero16 = jnp.zeros((16,), jnp.float32)
    zbuf = hrows[0]

    def zfill(i, carry):
        for k in range(_DH // 16):
            zbuf[i, pl.ds(k * 16, 16)] = zero16
        return carry

    pltpu.sync_copy(row_hbm.at[wid], row_v)
    pltpu.sync_copy(col_hbm.at[wid], col_v)

    ebase = wid * _EPW_A
    sl = pl.ds(sid * _ROWS_PER_SUB, _ROWS_PER_SUB)

    for p, h_hbm in enumerate((h0_hbm, h1_hbm)):
        lax.fori_loop(0, _ZROWS, zfill, 0)
        for t in range(_ROWS_PER_SUB // _ZROWS):
            pltpu.sync_copy(
                zbuf, agg_sh.at[pl.ds(sid * _ROWS_PER_SUB + t * _ZROWS, _ZROWS)])
        plsc.subcore_barrier()

        def fire(j, b):
            pltpu.async_copy(h_hbm.at[row_v.at[j]], hrows[b], gsem[b])
            pltpu.async_copy(
                rad_hbm.at[pl.ds(ebase + j * _CHUNK, _CHUNK),
                           pl.ds(p * _DH, _DH)],
                rad[b], rsem[b])

        def drain_gr(b):
            pltpu.make_async_copy(h_hbm.at[row_v.at[0]], hrows[b], gsem[b]).wait()
            pltpu.make_async_copy(
                rad_hbm.at[pl.ds(0, _CHUNK), pl.ds(p * _DH, _DH)],
                rad[b], rsem[b]).wait()

        def drain_sc(b):
            pltpu.make_async_copy(
                hrows[b], agg_sh.at[col_v.at[0]], ssem[b]).wait()

        fire(0, 0)
        fire(1, 1)

        def quad(jj, carry):
            for u in range(4):
                j = jj * 4 + u
                bn = (u + 2) % 4

                @pl.when(j >= 2)
                def _():
                    drain_sc(bn)

                @pl.when(j + 2 < _NCHUNK_A)
                def _():
                    fire(j + 2, bn)

                drain_gr(u)

                def mul(i, c2):
                    for k in range(_DH // 16):
                        s = pl.ds(k * 16, 16)
                        hrows[u][i, s] = hrows[u][i, s] * rad[u][i, s]
                    return c2

                lax.fori_loop(0, _CHUNK, mul, 0)
                pltpu.async_copy(hrows[u], agg_sh.at[col_v.at[j]], ssem[u],
                                 add=True)
            return carry

        lax.fori_loop(0, _NCHUNK_A // 4, quad, 0)
        drain_sc(2)
        drain_sc(3)
        plsc.subcore_barrier()

        pltpu.sync_copy(agg_sh.at[sl], out_hbm.at[cid, p, sl])
        plsc.subcore_barrier()


@jax.jit
def _sc_agg(h, rad1, row1, col1):
    mesh = plsc.VectorSubcoreMesh(core_axis_name="c", subcore_axis_name="s",
                                  num_cores=_NC, num_subcores=_NS)
    out = pl.kernel(
        _sc_body,
        out_type=jax.ShapeDtypeStruct((_NC, 2, _NPAD, _DH), jnp.float32),
        mesh=mesh,
        compiler_params=pltpu.CompilerParams(use_tc_tiling_on_sc=False),
        scratch_types=[
            pltpu.VMEM((_NCHUNK_A, _CHUNK), jnp.int32),
            pltpu.VMEM((_NCHUNK_A, _CHUNK), jnp.int32),
            [pltpu.VMEM((_CHUNK, _DH), jnp.float32) for _ in range(4)],
            [pltpu.VMEM((_CHUNK, _DH), jnp.float32) for _ in range(4)],
            pltpu.VMEM_SHARED((_NPAD, _DH), jnp.float32),
            [pltpu.SemaphoreType.DMA for _ in range(4)],
            [pltpu.SemaphoreType.DMA for _ in range(4)],
            [pltpu.SemaphoreType.DMA for _ in range(4)],
        ],
    )(h[:, :_DH], h[:, _DH:], rad1, row1, col1)
    outc = out[0] + out[1]
    return jnp.concatenate([outc[0], outc[1]], axis=-1)


_PW = 16

def _sc_pos_body(pos_hbm, idx_hbm, out_hbm, idx_v, gbuf, sem):
    cid = lax.axis_index("c")
    sid = lax.axis_index("s")
    wid = sid * _NC + cid
    for s in range(4):
        pltpu.sync_copy(idx_hbm.at[s, wid], idx_v)

        def fire(j, c):
            pltpu.async_copy(pos_hbm.at[idx_v.at[j]], gbuf.at[j], sem)
            return c

        lax.fori_loop(0, _NCHUNK_A, fire, 0)

        def drain(j, c):
            pltpu.make_async_copy(
                pos_hbm.at[idx_v.at[0]], gbuf.at[0], sem).wait()
            return c

        lax.fori_loop(0, _NCHUNK_A, drain, 0)
        pltpu.sync_copy(gbuf, out_hbm.at[s, wid])


@jax.jit
def _sc_pos(pos4, idx4):
    mesh = plsc.VectorSubcoreMesh(core_axis_name="c", subcore_axis_name="s",
                                  num_cores=_NC, num_subcores=_NS)
    out = pl.kernel(
        _sc_pos_body,
        out_type=jax.ShapeDtypeStruct((4, _NW, _NCHUNK_A, _CHUNK, _PW),
                                      jnp.float32),
        mesh=mesh,
        compiler_params=pltpu.CompilerParams(use_tc_tiling_on_sc=False),
        scratch_types=[
            pltpu.VMEM((_NCHUNK_A, _CHUNK), jnp.int32),
            pltpu.VMEM((_NCHUNK_A, _CHUNK, _PW), jnp.float32),
            pltpu.SemaphoreType.DMA,
        ],
    )(pos4, idx4)
    return out.reshape(4, _EPAD, _PW)


def _node_body(h_ref, agg_ref, w_ref, b_ref, o_ref):
    z = jnp.dot(h_ref[...] + agg_ref[...], w_ref[...],
                preferred_element_type=jnp.float32) + b_ref[...]
    o_ref[...] = jnp.where(z > 0, z, 0.01 * z)


def _node_update(h, agg, W, b):
    BLK = 400
    return pl.pallas_call(
        _node_body,
        grid=(N // BLK,),
        in_specs=[
            pl.BlockSpec((BLK, D), lambda i: (i, 0)),
            pl.BlockSpec((BLK, D), lambda i: (i, 0)),
            pl.BlockSpec((D, D), lambda i: (0, 0)),
            pl.BlockSpec((1, D), lambda i: (0, 0)),
        ],
        out_specs=pl.BlockSpec((BLK, D), lambda i: (i, 0)),
        out_shape=jax.ShapeDtypeStruct((N, D), jnp.float32),
    )(h, agg, W, b.reshape(1, D))


def _pad_idx(idx):
    pad = (jnp.arange(_EPAD - E, dtype=jnp.int32) * 41) % N
    return jnp.concatenate([idx, pad]).reshape(_NW, _NCHUNK_A, _CHUNK)


def kernel(x, pos, edge_index_intra, edge_index_inter, batch, params):
    h = jax.nn.silu(x @ params['W_in'] + params['b_in'])

    rows = [_pad_idx(edge_index_intra[0]), _pad_idx(edge_index_inter[0])]
    cols = [_pad_idx(edge_index_intra[1]), _pad_idx(edge_index_inter[1])]
    idx4 = jnp.stack([rows[0], cols[0], rows[1], cols[1]])
    pos4 = jnp.pad(pos, ((0, 0), (0, _PW - 3)))
    g = _sc_pos(pos4, idx4)
    rbfs = []
    for k in range(2):
        diff = g[2 * k, :E, :3] - g[2 * k + 1, :E, :3]
        dist = jnp.sqrt(jnp.sum(diff * diff, axis=-1) + 1e-12)
        rbfs.append(_rbf(dist))

    for l in range(3):
        hn = []
        for bi, tag in enumerate(('cov', 'ncov')):
            radial = jax.nn.silu(rbfs[bi] @ params[f'W_{tag}_coord{l}']
                                 + params[f'b_{tag}_coord{l}'])
            rad1 = jnp.pad(radial, ((0, _EPAD - E), (0, 0)))
            agg = _sc_agg(h, rad1, rows[bi], cols[bi])[:N]
            z = _node_update(h, agg, params[f'W_{tag}_node{l}'],
                             params[f'b_{tag}_node{l}'])
            hn.append(_bn(z, params[f'g_{tag}_bn{l}'], params[f'be_{tag}_bn{l}']))
        h = hn[0] + hn[1]

    h = jax.ops.segment_sum(h, batch, num_segments=G)
    for i in range(3):
        h = jax.nn.leaky_relu(h @ params[f'W_fc{i}'] + params[f'b_fc{i}'], 0.01)
        h = _bn(h, params[f'g_fc{i}'], params[f'be_fc{i}'])
    h = h @ params['W_fc3'] + params['b_fc3']
    return h.reshape(-1)
